# Initial kernel scaffold; baseline (speedup 1.0000x reference)
import jax
import jax.numpy as jnp
from jax.experimental import pallas as pl


def kernel(image):
    raise NotImplementedError("write your pallas kernel here")


# speedup vs baseline: 154.5810x; 154.5810x over previous
import functools

import jax
import jax.numpy as jnp
from jax import lax
from jax.experimental import pallas as pl
from jax.experimental.pallas import tpu as pltpu
from jax.experimental.pallas import tpu_sc as plsc

MIN_SZ = 33000
MAX_SZ = 34000
NBINS = 1024
NC, NS, L = 2, 16, 16
NW = NC * NS


@functools.lru_cache(maxsize=None)
def _build(n, chunk):
    per_w = n // NW
    nch = per_w // chunk
    mesh = plsc.VectorSubcoreMesh(
        core_axis_name="c", subcore_axis_name="s", num_cores=NC, num_subcores=NS
    )

    @functools.partial(
        pl.kernel,
        out_type=jax.ShapeDtypeStruct((NW * NBINS,), jnp.int32),
        mesh=mesh,
        compiler_params=pltpu.CompilerParams(needs_layout_passes=False),
        scratch_types=[
            pltpu.VMEM((chunk,), jnp.float32),
            pltpu.VMEM((L * NBINS,), jnp.int32),
            pltpu.VMEM((NBINS,), jnp.int32),
        ],
    )
    def hist_k(img_hbm, out_hbm, buf, subh, partial):
        wid = lax.axis_index("s") * NC + lax.axis_index("c")
        base = wid * per_w
        zero = jnp.zeros((L,), jnp.int32)

        def zbody(i, _):
            subh[pl.ds(i * L, L)] = zero
            return 0

        lax.fori_loop(0, (L * NBINS) // L, zbody, 0)

        ones = jnp.ones((L,), jnp.int32)
        laneoff = lax.iota(jnp.int32, L) * NBINS

        def cbody(c, _):
            pltpu.sync_copy(img_hbm.at[pl.ds(base + c * chunk, chunk)], buf)

            def vbody(i, _):
                v = buf[pl.ds(i * L, L)]
                idx = v.astype(jnp.int32) + laneoff
                plsc.addupdate_scatter(subh, [idx], ones)
                return 0

            lax.fori_loop(0, chunk // L, vbody, 0)
            return 0

        lax.fori_loop(0, nch, cbody, 0)

        def rbody(j, _):
            acc = subh[pl.ds(j * L, L)]
            for lane in range(1, L):
                acc = acc + subh[pl.ds(lane * NBINS + j * L, L)]
            partial[pl.ds(j * L, L)] = acc
            return 0

        lax.fori_loop(0, NBINS // L, rbody, 0)
        pltpu.sync_copy(partial, out_hbm.at[pl.ds(wid * NBINS, NBINS)])

    @functools.partial(
        pl.kernel,
        out_type=jax.ShapeDtypeStruct((n,), jnp.float32),
        mesh=mesh,
        compiler_params=pltpu.CompilerParams(needs_layout_passes=False),
        scratch_types=[
            pltpu.VMEM((NW * NBINS,), jnp.int32),
            pltpu.VMEM((NBINS,), jnp.float32),
            pltpu.VMEM((chunk,), jnp.float32),
        ],
    )
    def mask_k(img_hbm, hist_hbm, out_hbm, parts, table, buf):
        wid = lax.axis_index("s") * NC + lax.axis_index("c")
        base = wid * per_w
        pltpu.sync_copy(hist_hbm, parts)
        lane_i = lax.iota(jnp.int32, L)

        def tbody(j, _):
            acc = parts[pl.ds(j * L, L)]
            for w in range(1, NW):
                acc = acc + parts[pl.ds(w * NBINS + j * L, L)]
            lvals = (lane_i + j * L).astype(jnp.float32)
            keep = (acc >= MIN_SZ) & (acc <= MAX_SZ)
            table[pl.ds(j * L, L)] = jnp.where(keep, lvals, jnp.float32(0.0))
            return 0

        lax.fori_loop(0, NBINS // L, tbody, 0)

        def cbody(c, _):
            pltpu.sync_copy(img_hbm.at[pl.ds(base + c * chunk, chunk)], buf)

            def vbody(i, _):
                idx = buf[pl.ds(i * L, L)].astype(jnp.int32)
                buf[pl.ds(i * L, L)] = plsc.load_gather(table, [idx])
                return 0

            lax.fori_loop(0, chunk // L, vbody, 0)
            pltpu.sync_copy(buf, out_hbm.at[pl.ds(base + c * chunk, chunk)])
            return 0

        lax.fori_loop(0, nch, cbody, 0)

    return hist_k, mask_k


def kernel(image):
    n = image.size
    chunk = 32768
    while n % (NW * chunk) != 0:
        chunk //= 2
    flat = image.reshape(-1)
    hist_k, mask_k = _build(n, chunk)
    parts = hist_k(flat)
    out = mask_k(flat, parts)
    return out.reshape(image.shape)


# speedup vs baseline: 587.4885x; 3.8005x over previous
import functools

import jax
import jax.numpy as jnp
from jax import lax
from jax.experimental import pallas as pl
from jax.experimental.pallas import tpu as pltpu
from jax.experimental.pallas import tpu_sc as plsc

MIN_SZ = 33000
MAX_SZ = 34000
NBINS = 1024
NC, NS, L = 2, 16, 16
NW = NC * NS


@functools.lru_cache(maxsize=None)
def _build(n, chunk1, chunk2):
    per_w = n // NW
    nch1 = per_w // chunk1
    nch2 = per_w // chunk2
    mesh = plsc.VectorSubcoreMesh(
        core_axis_name="c", subcore_axis_name="s", num_cores=NC, num_subcores=NS
    )

    @functools.partial(
        pl.kernel,
        out_type=jax.ShapeDtypeStruct((NW * NBINS,), jnp.int32),
        mesh=mesh,
        compiler_params=pltpu.CompilerParams(needs_layout_passes=False),
        scratch_types=[
            pltpu.VMEM((chunk1,), jnp.float32),
            pltpu.VMEM((chunk1,), jnp.float32),
            pltpu.VMEM((L * NBINS,), jnp.int32),
            pltpu.VMEM((NBINS,), jnp.int32),
            pltpu.SemaphoreType.DMA,
            pltpu.SemaphoreType.DMA,
        ],
    )
    def hist_k(img_hbm, out_hbm, bufa, bufb, subh, partial, sema, semb):
        wid = lax.axis_index("s") * NC + lax.axis_index("c")
        base = wid * per_w
        bufs = (bufa, bufb)
        sems = (sema, semb)
        zero = jnp.zeros((L,), jnp.int32)

        def zbody(i, _):
            subh[pl.ds(i * L, L)] = zero
            return 0

        lax.fori_loop(0, (L * NBINS) // L, zbody, 0)

        ones = jnp.ones((L,), jnp.int32)
        laneoff = lax.iota(jnp.int32, L) * NBINS

        def src(c):
            return img_hbm.at[pl.ds(base + c * chunk1, chunk1)]

        pltpu.async_copy(src(0), bufa, sema)
        pltpu.async_copy(src(1), bufb, semb)

        def pbody(p, _):
            for b in range(2):
                c = 2 * p + b
                buf, sem = bufs[b], sems[b]
                pltpu.make_async_copy(src(c), buf, sem).wait()

                @plsc.parallel_loop(0, chunk1 // L, 1, unroll=8)
                def _(i):
                    v = buf[pl.ds(i * L, L)]
                    idx = v.astype(jnp.int32) + laneoff
                    plsc.addupdate_scatter(subh, [idx], ones)

                @pl.when(c + 2 < nch1)
                def _():
                    pltpu.async_copy(src(c + 2), buf, sem)

            return 0

        lax.fori_loop(0, nch1 // 2, pbody, 0)

        def rbody(j, _):
            acc = subh[pl.ds(j * L, L)]
            for lane in range(1, L):
                acc = acc + subh[pl.ds(lane * NBINS + j * L, L)]
            partial[pl.ds(j * L, L)] = acc
            return 0

        lax.fori_loop(0, NBINS // L, rbody, 0)
        pltpu.sync_copy(partial, out_hbm.at[pl.ds(wid * NBINS, NBINS)])

    @functools.partial(
        pl.kernel,
        out_type=jax.ShapeDtypeStruct((n,), jnp.float32),
        mesh=mesh,
        compiler_params=pltpu.CompilerParams(needs_layout_passes=False),
        scratch_types=[
            pltpu.VMEM((NW * NBINS,), jnp.int32),
            pltpu.VMEM((NBINS,), jnp.float32),
            pltpu.VMEM((chunk2,), jnp.float32),
            pltpu.VMEM((chunk2,), jnp.float32),
            pltpu.VMEM((chunk2,), jnp.float32),
            pltpu.VMEM((chunk2,), jnp.float32),
            pltpu.SemaphoreType.DMA,
            pltpu.SemaphoreType.DMA,
            pltpu.SemaphoreType.DMA,
            pltpu.SemaphoreType.DMA,
        ],
    )
    def mask_k(img_hbm, hist_hbm, out_hbm, parts, table, ina, inb, outa, outb,
               sia, sib, soa, sob):
        wid = lax.axis_index("s") * NC + lax.axis_index("c")
        base = wid * per_w
        ibufs, obufs = (ina, inb), (outa, outb)
        isems, osems = (sia, sib), (soa, sob)

        def src(c):
            return img_hbm.at[pl.ds(base + c * chunk2, chunk2)]

        def dst(c):
            return out_hbm.at[pl.ds(base + c * chunk2, chunk2)]

        pltpu.async_copy(src(0), ina, sia)
        pltpu.async_copy(src(1), inb, sib)

        pltpu.sync_copy(hist_hbm, parts)
        lane_i = lax.iota(jnp.int32, L)

        def tbody(j, _):
            acc = parts[pl.ds(j * L, L)]
            for w in range(1, NW):
                acc = acc + parts[pl.ds(w * NBINS + j * L, L)]
            lvals = (lane_i + j * L).astype(jnp.float32)
            keep = (acc >= MIN_SZ) & (acc <= MAX_SZ)
            table[pl.ds(j * L, L)] = jnp.where(keep, lvals, jnp.float32(0.0))
            return 0

        lax.fori_loop(0, NBINS // L, tbody, 0)

        def pbody(p, _):
            for b in range(2):
                c = 2 * p + b
                ib, ob = ibufs[b], obufs[b]
                isem, osem = isems[b], osems[b]
                pltpu.make_async_copy(src(c), ib, isem).wait()

                @pl.when(c >= 2)
                def _():
                    pltpu.make_async_copy(ob, dst(c - 2), osem).wait()

                @plsc.parallel_loop(0, chunk2 // L, 1, unroll=8)
                def _(i):
                    idx = ib[pl.ds(i * L, L)].astype(jnp.int32)
                    ob[pl.ds(i * L, L)] = plsc.load_gather(table, [idx])

                pltpu.async_copy(ob, dst(c), osem)

                @pl.when(c + 2 < nch2)
                def _():
                    pltpu.async_copy(src(c + 2), ib, isem)

            return 0

        lax.fori_loop(0, nch2 // 2, pbody, 0)
        pltpu.make_async_copy(outa, dst(nch2 - 2), soa).wait()
        pltpu.make_async_copy(outb, dst(nch2 - 1), sob).wait()

    return hist_k, mask_k


def kernel(image):
    n = image.size
    chunk1 = 32768
    while n % (NW * chunk1 * 2) != 0:
        chunk1 //= 2
    chunk2 = chunk1 // 2
    flat = image.reshape(-1)
    hist_k, mask_k = _build(n, chunk1, chunk2)
    parts = hist_k(flat)
    out = mask_k(flat, parts)
    return out.reshape(image.shape)


# speedup vs baseline: 1060.3594x; 1.8049x over previous
import functools

import jax
import jax.numpy as jnp
from jax import lax
from jax.experimental import pallas as pl
from jax.experimental.pallas import tpu as pltpu
from jax.experimental.pallas import tpu_sc as plsc

MIN_SZ = 33000
MAX_SZ = 34000
NBINS = 1024
NC, NS, L = 2, 16, 16
NW = NC * NS

_CP = pltpu.CompilerParams(needs_layout_passes=False, use_tc_tiling_on_sc=True)


@functools.lru_cache(maxsize=None)
def _build(d0, d1, d2, rows1, rows2):
    slabs_per_w = d0 // NW
    nch1 = slabs_per_w * (d1 // rows1)
    nch2 = slabs_per_w * (d1 // rows2)
    ch1_per_slab = d1 // rows1
    ch2_per_slab = d1 // rows2
    mesh = plsc.VectorSubcoreMesh(
        core_axis_name="c", subcore_axis_name="s", num_cores=NC, num_subcores=NS
    )

    @functools.partial(
        pl.kernel,
        out_type=jax.ShapeDtypeStruct((NW * NBINS,), jnp.int32),
        mesh=mesh,
        compiler_params=_CP,
        scratch_types=[
            pltpu.VMEM((rows1, d2), jnp.float32),
            pltpu.VMEM((rows1, d2), jnp.float32),
            pltpu.VMEM((L * NBINS,), jnp.int32),
            pltpu.VMEM((NBINS,), jnp.int32),
            pltpu.SemaphoreType.DMA,
            pltpu.SemaphoreType.DMA,
        ],
    )
    def hist_k(img_hbm, out_hbm, bufa, bufb, subh, partial, sema, semb):
        wid = lax.axis_index("s") * NC + lax.axis_index("c")
        bufs = (bufa, bufb)
        sems = (sema, semb)
        zero = jnp.zeros((L,), jnp.int32)

        def zbody(i, _):
            subh[pl.ds(i * L, L)] = zero
            return 0

        lax.fori_loop(0, (L * NBINS) // L, zbody, 0)

        ones = jnp.ones((L,), jnp.int32)
        laneoff = lax.iota(jnp.int32, L) * NBINS

        def src(c):
            s = wid * slabs_per_w + c // ch1_per_slab
            r = (c % ch1_per_slab) * rows1
            return img_hbm.at[s, pl.ds(r, rows1)]

        pltpu.async_copy(src(0), bufa, sema)
        pltpu.async_copy(src(1), bufb, semb)

        def pbody(p, _):
            for b in range(2):
                c = 2 * p + b
                buf, sem = bufs[b], sems[b]
                pltpu.make_async_copy(src(c), buf, sem).wait()

                @plsc.parallel_loop(0, (rows1 * d2) // L, 1, unroll=8)
                def _(i):
                    v = buf[i // (d2 // L), pl.ds((i % (d2 // L)) * L, L)]
                    idx = v.astype(jnp.int32) + laneoff
                    plsc.addupdate_scatter(subh, [idx], ones)

                @pl.when(c + 2 < nch1)
                def _():
                    pltpu.async_copy(src(c + 2), buf, sem)

            return 0

        lax.fori_loop(0, nch1 // 2, pbody, 0)

        def rbody(j, _):
            acc = subh[pl.ds(j * L, L)]
            for lane in range(1, L):
                acc = acc + subh[pl.ds(lane * NBINS + j * L, L)]
            partial[pl.ds(j * L, L)] = acc
            return 0

        lax.fori_loop(0, NBINS // L, rbody, 0)
        pltpu.sync_copy(partial, out_hbm.at[pl.ds(wid * NBINS, NBINS)])

    @functools.partial(
        pl.kernel,
        out_type=jax.ShapeDtypeStruct((d0, d1, d2), jnp.float32),
        mesh=mesh,
        compiler_params=_CP,
        scratch_types=[
            pltpu.VMEM((NW * NBINS,), jnp.int32),
            pltpu.VMEM((NBINS,), jnp.float32),
            pltpu.VMEM((rows2, d2), jnp.float32),
            pltpu.VMEM((rows2, d2), jnp.float32),
            pltpu.VMEM((rows2, d2), jnp.float32),
            pltpu.VMEM((rows2, d2), jnp.float32),
            pltpu.SemaphoreType.DMA,
            pltpu.SemaphoreType.DMA,
            pltpu.SemaphoreType.DMA,
            pltpu.SemaphoreType.DMA,
        ],
    )
    def mask_k(img_hbm, hist_hbm, out_hbm, parts, table, ina, inb, outa, outb,
               sia, sib, soa, sob):
        wid = lax.axis_index("s") * NC + lax.axis_index("c")
        ibufs, obufs = (ina, inb), (outa, outb)
        isems, osems = (sia, sib), (soa, sob)

        def loc(c):
            s = wid * slabs_per_w + c // ch2_per_slab
            r = (c % ch2_per_slab) * rows2
            return s, r

        def src(c):
            s, r = loc(c)
            return img_hbm.at[s, pl.ds(r, rows2)]

        def dst(c):
            s, r = loc(c)
            return out_hbm.at[s, pl.ds(r, rows2)]

        pltpu.async_copy(src(0), ina, sia)
        pltpu.async_copy(src(1), inb, sib)

        pltpu.sync_copy(hist_hbm, parts)
        lane_i = lax.iota(jnp.int32, L)

        def tbody(j, _):
            acc = parts[pl.ds(j * L, L)]
            for w in range(1, NW):
                acc = acc + parts[pl.ds(w * NBINS + j * L, L)]
            lvals = (lane_i + j * L).astype(jnp.float32)
            keep = (acc >= MIN_SZ) & (acc <= MAX_SZ)
            table[pl.ds(j * L, L)] = jnp.where(keep, lvals, jnp.float32(0.0))
            return 0

        lax.fori_loop(0, NBINS // L, tbody, 0)

        def pbody(p, _):
            for b in range(2):
                c = 2 * p + b
                ib, ob = ibufs[b], obufs[b]
                isem, osem = isems[b], osems[b]
                pltpu.make_async_copy(src(c), ib, isem).wait()

                @pl.when(c >= 2)
                def _():
                    pltpu.make_async_copy(ob, dst(c - 2), osem).wait()

                @plsc.parallel_loop(0, (rows2 * d2) // L, 1, unroll=8)
                def _(i):
                    r = i // (d2 // L)
                    col = (i % (d2 // L)) * L
                    idx = ib[r, pl.ds(col, L)].astype(jnp.int32)
                    ob[r, pl.ds(col, L)] = plsc.load_gather(table, [idx])

                pltpu.async_copy(ob, dst(c), osem)

                @pl.when(c + 2 < nch2)
                def _():
                    pltpu.async_copy(src(c + 2), ib, isem)

            return 0

        lax.fori_loop(0, nch2 // 2, pbody, 0)
        pltpu.make_async_copy(outa, dst(nch2 - 2), soa).wait()
        pltpu.make_async_copy(outb, dst(nch2 - 1), sob).wait()

    return hist_k, mask_k


def kernel(image):
    d0, d1, d2 = image.shape
    rows1 = 64
    while d1 % (rows1 * 2) != 0:
        rows1 //= 2
    rows2 = rows1 // 2
    hist_k, mask_k = _build(d0, d1, d2, rows1, rows2)
    parts = hist_k(image)
    return mask_k(image, parts)


# speedup vs baseline: 1363.0117x; 1.2854x over previous
import functools

import jax
import jax.numpy as jnp
from jax import lax
from jax.experimental import pallas as pl
from jax.experimental.pallas import tpu as pltpu
from jax.experimental.pallas import tpu_sc as plsc

MIN_SZ = 33000
MAX_SZ = 34000
NBINS = 1024
NC, NS, L = 2, 16, 16
NW = NC * NS

_CP = pltpu.CompilerParams(needs_layout_passes=False, use_tc_tiling_on_sc=True)


@functools.lru_cache(maxsize=None)
def _build(d0, d1, d2, rows1, rows2):
    slabs_per_w = d0 // NW
    nch1 = slabs_per_w * (d1 // rows1)
    nch2 = slabs_per_w * (d1 // rows2)
    ch1_per_slab = d1 // rows1
    ch2_per_slab = d1 // rows2
    mesh = plsc.VectorSubcoreMesh(
        core_axis_name="c", subcore_axis_name="s", num_cores=NC, num_subcores=NS
    )

    @functools.partial(
        pl.kernel,
        out_type=jax.ShapeDtypeStruct((NW * NBINS,), jnp.int32),
        mesh=mesh,
        compiler_params=_CP,
        scratch_types=[
            pltpu.VMEM((rows1, d2), jnp.float32),
            pltpu.VMEM((rows1, d2), jnp.float32),
            pltpu.VMEM((NBINS * L,), jnp.int32),
            pltpu.VMEM((NBINS,), jnp.int32),
            pltpu.SemaphoreType.DMA,
            pltpu.SemaphoreType.DMA,
        ],
    )
    def hist_k(img_hbm, out_hbm, bufa, bufb, subh, partial, sema, semb):
        wid = lax.axis_index("s") * NC + lax.axis_index("c")
        bufs = (bufa, bufb)
        sems = (sema, semb)
        zero = jnp.zeros((L,), jnp.int32)

        def zbody(i, _):
            subh[pl.ds(i * L, L)] = zero
            return 0

        lax.fori_loop(0, NBINS, zbody, 0)

        ones = jnp.ones((L,), jnp.int32)
        lane_f = lax.iota(jnp.int32, L).astype(jnp.float32)
        sixteen = jnp.float32(L)

        def src(c):
            s = wid * slabs_per_w + c // ch1_per_slab
            r = (c % ch1_per_slab) * rows1
            return img_hbm.at[s, pl.ds(r, rows1)]

        pltpu.async_copy(src(0), bufa, sema)
        pltpu.async_copy(src(1), bufb, semb)

        def pbody(p, _):
            for b in range(2):
                c = 2 * p + b
                buf, sem = bufs[b], sems[b]
                pltpu.make_async_copy(src(c), buf, sem).wait()

                @plsc.parallel_loop(0, (rows1 * d2) // L, 1, unroll=16)
                def _(i):
                    v = buf[i // (d2 // L), pl.ds((i % (d2 // L)) * L, L)]
                    idx = (v * sixteen + lane_f).astype(jnp.int32)
                    plsc.addupdate_scatter(subh, [idx], ones)

                @pl.when(c + 2 < nch1)
                def _():
                    pltpu.async_copy(src(c + 2), buf, sem)

            return 0

        lax.fori_loop(0, nch1 // 2, pbody, 0)

        lane_i = lax.iota(jnp.int32, L)

        def rbody(j, _):
            res = jnp.zeros((L,), jnp.int32)
            for k in range(L):
                row = subh[pl.ds((j * L + k) * L, L)]
                s = jnp.sum(row)
                res = jnp.where(lane_i == k, s, res)
            partial[pl.ds(j * L, L)] = res
            return 0

        lax.fori_loop(0, NBINS // L, rbody, 0)
        pltpu.sync_copy(partial, out_hbm.at[pl.ds(wid * NBINS, NBINS)])

    @functools.partial(
        pl.kernel,
        out_type=jax.ShapeDtypeStruct((d0, d1, d2), jnp.float32),
        mesh=mesh,
        compiler_params=_CP,
        scratch_types=[
            pltpu.VMEM((NW * NBINS,), jnp.int32),
            pltpu.VMEM((NBINS * L,), jnp.float32),
            pltpu.VMEM((rows2, d2), jnp.float32),
            pltpu.VMEM((rows2, d2), jnp.float32),
            pltpu.VMEM((rows2, d2), jnp.float32),
            pltpu.VMEM((rows2, d2), jnp.float32),
            pltpu.SemaphoreType.DMA,
            pltpu.SemaphoreType.DMA,
            pltpu.SemaphoreType.DMA,
            pltpu.SemaphoreType.DMA,
        ],
    )
    def mask_k(img_hbm, hist_hbm, out_hbm, parts, table2, ina, inb, outa, outb,
               sia, sib, soa, sob):
        wid = lax.axis_index("s") * NC + lax.axis_index("c")
        ibufs, obufs = (ina, inb), (outa, outb)
        isems, osems = (sia, sib), (soa, sob)

        def loc(c):
            s = wid * slabs_per_w + c // ch2_per_slab
            r = (c % ch2_per_slab) * rows2
            return s, r

        def src(c):
            s, r = loc(c)
            return img_hbm.at[s, pl.ds(r, rows2)]

        def dst(c):
            s, r = loc(c)
            return out_hbm.at[s, pl.ds(r, rows2)]

        pltpu.async_copy(src(0), ina, sia)
        pltpu.async_copy(src(1), inb, sib)

        pltpu.sync_copy(hist_hbm, parts)
        lane_i = lax.iota(jnp.int32, L)
        lane_f = lane_i.astype(jnp.float32)
        sixteen = jnp.float32(L)

        def tbody(j, _):
            acc = parts[pl.ds(j * L, L)]
            for w in range(1, NW):
                acc = acc + parts[pl.ds(w * NBINS + j * L, L)]
            lvals = (lane_i + j * L).astype(jnp.float32)
            keep = (acc >= MIN_SZ) & (acc <= MAX_SZ)
            t = jnp.where(keep, lvals, jnp.float32(0.0))
            for k in range(L):
                s = jnp.sum(jnp.where(lane_i == k, t, jnp.float32(0.0)))
                table2[pl.ds((j * L + k) * L, L)] = jnp.full((L,), 0.0, jnp.float32) + s
            return 0

        lax.fori_loop(0, NBINS // L, tbody, 0)

        def pbody(p, _):
            for b in range(2):
                c = 2 * p + b
                ib, ob = ibufs[b], obufs[b]
                isem, osem = isems[b], osems[b]
                pltpu.make_async_copy(src(c), ib, isem).wait()

                @pl.when(c >= 2)
                def _():
                    pltpu.make_async_copy(ob, dst(c - 2), osem).wait()

                @plsc.parallel_loop(0, (rows2 * d2) // L, 1, unroll=16)
                def _(i):
                    r = i // (d2 // L)
                    col = (i % (d2 // L)) * L
                    v = ib[r, pl.ds(col, L)]
                    idx = (v * sixteen + lane_f).astype(jnp.int32)
                    ob[r, pl.ds(col, L)] = plsc.load_gather(table2, [idx])

                pltpu.async_copy(ob, dst(c), osem)

                @pl.when(c + 2 < nch2)
                def _():
                    pltpu.async_copy(src(c + 2), ib, isem)

            return 0

        lax.fori_loop(0, nch2 // 2, pbody, 0)
        pltpu.make_async_copy(outa, dst(nch2 - 2), soa).wait()
        pltpu.make_async_copy(outb, dst(nch2 - 1), sob).wait()

    return hist_k, mask_k


def kernel(image):
    d0, d1, d2 = image.shape
    rows1 = 64
    while d1 % (rows1 * 2) != 0:
        rows1 //= 2
    rows2 = rows1 // 2
    hist_k, mask_k = _build(d0, d1, d2, rows1, rows2)
    parts = hist_k(image)
    return mask_k(image, parts)


# speedup vs baseline: 1423.2584x; 1.0442x over previous
import functools

import jax
import jax.numpy as jnp
from jax import lax
from jax.experimental import pallas as pl
from jax.experimental.pallas import tpu as pltpu
from jax.experimental.pallas import tpu_sc as plsc

MIN_SZ = 33000
MAX_SZ = 34000
NBINS = 1024
NC, NS, L = 2, 16, 16
NW = NC * NS

_CP = pltpu.CompilerParams(needs_layout_passes=False, use_tc_tiling_on_sc=True)


@functools.lru_cache(maxsize=None)
def _build(d0, d1, d2, rows1, rows2):
    slabs_per_w = d0 // NW
    nch1 = slabs_per_w * (d1 // rows1)
    nch2 = slabs_per_w * (d1 // rows2)
    ch1_per_slab = d1 // rows1
    ch2_per_slab = d1 // rows2
    chunk1 = rows1 * d2
    chunk2 = rows2 * d2
    per_w = slabs_per_w * d1 * d2
    n = d0 * d1 * d2
    mesh = plsc.VectorSubcoreMesh(
        core_axis_name="c", subcore_axis_name="s", num_cores=NC, num_subcores=NS
    )

    @functools.partial(
        pl.kernel,
        out_type=(
            jax.ShapeDtypeStruct((NW * NBINS,), jnp.int32),
            jax.ShapeDtypeStruct((n // 2,), jnp.int32),
        ),
        mesh=mesh,
        compiler_params=_CP,
        scratch_types=[
            pltpu.VMEM((rows1, d2), jnp.float32),
            pltpu.VMEM((rows1, d2), jnp.float32),
            pltpu.VMEM((chunk1 // 2,), jnp.int32),
            pltpu.VMEM((chunk1 // 2,), jnp.int32),
            pltpu.VMEM((NBINS * L,), jnp.int32),
            pltpu.VMEM((NBINS,), jnp.int32),
            pltpu.SemaphoreType.DMA,
            pltpu.SemaphoreType.DMA,
            pltpu.SemaphoreType.DMA,
            pltpu.SemaphoreType.DMA,
        ],
    )
    def hist_k(img_hbm, out_hbm, idx_hbm, bufa, bufb, ixa, ixb, subh, partial,
               sema, semb, sxa, sxb):
        wid = lax.axis_index("s") * NC + lax.axis_index("c")
        bufs = (bufa, bufb)
        ixbufs = (ixa, ixb)
        sems = (sema, semb)
        xsems = (sxa, sxb)
        zero = jnp.zeros((L,), jnp.int32)

        def zbody(i, _):
            subh[pl.ds(i * L, L)] = zero
            return 0

        lax.fori_loop(0, NBINS, zbody, 0)

        ones = jnp.ones((L,), jnp.int32)
        lane_f = lax.iota(jnp.int32, L).astype(jnp.float32)
        sixteen = jnp.float32(L)

        def src(c):
            s = wid * slabs_per_w + c // ch1_per_slab
            r = (c % ch1_per_slab) * rows1
            return img_hbm.at[s, pl.ds(r, rows1)]

        def xdst(c):
            return idx_hbm.at[pl.ds(wid * (per_w // 2) + c * (chunk1 // 2), chunk1 // 2)]

        pltpu.async_copy(src(0), bufa, sema)
        pltpu.async_copy(src(1), bufb, semb)

        nvp = d2 // L

        def pbody(p, _):
            for b in range(2):
                c = 2 * p + b
                buf, sem = bufs[b], sems[b]
                ix, xsem = ixbufs[b], xsems[b]
                pltpu.make_async_copy(src(c), buf, sem).wait()

                @pl.when(c >= 2)
                def _():
                    pltpu.make_async_copy(ix, xdst(c - 2), xsem).wait()

                @plsc.parallel_loop(0, chunk1 // (2 * L), 1, unroll=16)
                def _(i):
                    q0 = 2 * i
                    q1 = 2 * i + 1
                    va = buf[q0 // nvp, pl.ds((q0 % nvp) * L, L)]
                    vb = buf[q1 // nvp, pl.ds((q1 % nvp) * L, L)]
                    ia = (va * sixteen + lane_f).astype(jnp.int32)
                    ib = (vb * sixteen + lane_f).astype(jnp.int32)
                    plsc.addupdate_scatter(subh, [ia], ones)
                    plsc.addupdate_scatter(subh, [ib], ones)
                    packed = plsc.pack(ia, ib, format=plsc.PackFormat.INTERLEAVED)
                    ix[pl.ds(pl.multiple_of(i * L, L), L)] = plsc.bitcast(
                        packed, jnp.int32
                    )

                pltpu.async_copy(ix, xdst(c), xsem)

                @pl.when(c + 2 < nch1)
                def _():
                    pltpu.async_copy(src(c + 2), buf, sem)

            return 0

        lax.fori_loop(0, nch1 // 2, pbody, 0)
        pltpu.make_async_copy(ixa, xdst(nch1 - 2), sxa).wait()
        pltpu.make_async_copy(ixb, xdst(nch1 - 1), sxb).wait()

        lane_i = lax.iota(jnp.int32, L)

        def rbody(j, _):
            res = jnp.zeros((L,), jnp.int32)
            for k in range(L):
                row = subh[pl.ds((j * L + k) * L, L)]
                s = jnp.sum(row)
                res = jnp.where(lane_i == k, s, res)
            partial[pl.ds(j * L, L)] = res
            return 0

        lax.fori_loop(0, NBINS // L, rbody, 0)
        pltpu.sync_copy(partial, out_hbm.at[pl.ds(wid * NBINS, NBINS)])

    @functools.partial(
        pl.kernel,
        out_type=jax.ShapeDtypeStruct((d0, d1, d2), jnp.float32),
        mesh=mesh,
        compiler_params=_CP,
        scratch_types=[
            pltpu.VMEM((NW * NBINS,), jnp.int32),
            pltpu.VMEM((NBINS * L,), jnp.float32),
            pltpu.VMEM((NBINS,), jnp.float32),
            pltpu.VMEM((chunk2 // 2,), jnp.int32),
            pltpu.VMEM((chunk2 // 2,), jnp.int32),
            pltpu.VMEM((rows2, d2), jnp.float32),
            pltpu.VMEM((rows2, d2), jnp.float32),
            pltpu.SemaphoreType.DMA,
            pltpu.SemaphoreType.DMA,
            pltpu.SemaphoreType.DMA,
            pltpu.SemaphoreType.DMA,
        ],
    )
    def mask_k(idx_hbm, hist_hbm, out_hbm, parts, table2, staging, ina, inb,
               outa, outb, sia, sib, soa, sob):
        wid = lax.axis_index("s") * NC + lax.axis_index("c")
        ibufs, obufs = (ina, inb), (outa, outb)
        isems, osems = (sia, sib), (soa, sob)

        def src(c):
            return idx_hbm.at[pl.ds(wid * (per_w // 2) + c * (chunk2 // 2), chunk2 // 2)]

        def dst(c):
            s = wid * slabs_per_w + c // ch2_per_slab
            r = (c % ch2_per_slab) * rows2
            return out_hbm.at[s, pl.ds(r, rows2)]

        pltpu.async_copy(src(0), ina, sia)
        pltpu.async_copy(src(1), inb, sib)

        pltpu.sync_copy(hist_hbm, parts)
        lane_i = lax.iota(jnp.int32, L)

        def tbody(j, _):
            acc = parts[pl.ds(j * L, L)]
            for w in range(1, NW):
                acc = acc + parts[pl.ds(w * NBINS + j * L, L)]
            lvals = (lane_i + j * L).astype(jnp.float32)
            keep = (acc >= MIN_SZ) & (acc <= MAX_SZ)
            staging[pl.ds(j * L, L)] = jnp.where(keep, lvals, jnp.float32(0.0))
            return 0

        lax.fori_loop(0, NBINS // L, tbody, 0)

        def t2body(j, _):
            t = staging[pl.ds(j * L, L)]
            for k in range(L):
                table2[pl.ds((j * L + k) * L, L)] = jnp.full((L,), t[k], jnp.float32)
            return 0

        lax.fori_loop(0, NBINS // L, t2body, 0)

        nvp = d2 // L

        def pbody(p, _):
            for b in range(2):
                c = 2 * p + b
                ib, ob = ibufs[b], obufs[b]
                isem, osem = isems[b], osems[b]
                pltpu.make_async_copy(src(c), ib, isem).wait()

                @pl.when(c >= 2)
                def _():
                    pltpu.make_async_copy(ob, dst(c - 2), osem).wait()

                @plsc.parallel_loop(0, chunk2 // (2 * L), 1, unroll=16)
                def _(i):
                    packed = plsc.bitcast(
                        ib[pl.ds(pl.multiple_of(i * L, L), L)], jnp.int16
                    )
                    ia, ixb = plsc.unpack(packed, format=plsc.PackFormat.INTERLEAVED)
                    q0 = 2 * i
                    q1 = 2 * i + 1
                    ob[q0 // nvp, pl.ds((q0 % nvp) * L, L)] = plsc.load_gather(
                        table2, [ia]
                    )
                    ob[q1 // nvp, pl.ds((q1 % nvp) * L, L)] = plsc.load_gather(
                        table2, [ixb]
                    )

                pltpu.async_copy(ob, dst(c), osem)

                @pl.when(c + 2 < nch2)
                def _():
                    pltpu.async_copy(src(c + 2), ib, isem)

            return 0

        lax.fori_loop(0, nch2 // 2, pbody, 0)
        pltpu.make_async_copy(outa, dst(nch2 - 2), soa).wait()
        pltpu.make_async_copy(outb, dst(nch2 - 1), sob).wait()

    return hist_k, mask_k


def kernel(image):
    d0, d1, d2 = image.shape
    rows1 = 64
    while d1 % (rows1 * 2) != 0:
        rows1 //= 2
    rows2 = rows1 // 2
    hist_k, mask_k = _build(d0, d1, d2, rows1, rows2)
    parts, idx16 = hist_k(image)
    return mask_k(idx16, parts)


# speedup vs baseline: 1449.1878x; 1.0182x over previous
import functools

import jax
import jax.numpy as jnp
from jax import lax
from jax.experimental import pallas as pl
from jax.experimental.pallas import tpu as pltpu
from jax.experimental.pallas import tpu_sc as plsc

MIN_SZ = 33000
MAX_SZ = 34000
NBINS = 1024
NC, NS, L = 2, 16, 16
NW = NC * NS

_CP = pltpu.CompilerParams(needs_layout_passes=False, use_tc_tiling_on_sc=True)


@functools.lru_cache(maxsize=None)
def _build(d0, d1, d2, rows1, rows2):
    slabs_per_w = d0 // NW
    nch1 = slabs_per_w * (d1 // rows1)
    nch2 = slabs_per_w * (d1 // rows2)
    ch1_per_slab = d1 // rows1
    ch2_per_slab = d1 // rows2
    chunk1 = rows1 * d2
    chunk2 = rows2 * d2
    per_w = slabs_per_w * d1 * d2
    n = d0 * d1 * d2
    mesh = plsc.VectorSubcoreMesh(
        core_axis_name="c", subcore_axis_name="s", num_cores=NC, num_subcores=NS
    )

    @functools.partial(
        pl.kernel,
        out_type=(
            jax.ShapeDtypeStruct((NW * NBINS,), jnp.int32),
            jax.ShapeDtypeStruct((n // 2,), jnp.int32),
        ),
        mesh=mesh,
        compiler_params=_CP,
        scratch_types=[
            pltpu.VMEM((rows1, d2), jnp.float32),
            pltpu.VMEM((rows1, d2), jnp.float32),
            pltpu.VMEM((chunk1 // 2,), jnp.int32),
            pltpu.VMEM((chunk1 // 2,), jnp.int32),
            pltpu.VMEM((NBINS * L,), jnp.int32),
            pltpu.VMEM((NBINS,), jnp.int32),
            pltpu.SemaphoreType.DMA,
            pltpu.SemaphoreType.DMA,
            pltpu.SemaphoreType.DMA,
            pltpu.SemaphoreType.DMA,
        ],
    )
    def hist_k(img_hbm, out_hbm, idx_hbm, bufa, bufb, ixa, ixb, subh, partial,
               sema, semb, sxa, sxb):
        wid = lax.axis_index("s") * NC + lax.axis_index("c")
        bufs = (bufa, bufb)
        ixbufs = (ixa, ixb)
        sems = (sema, semb)
        xsems = (sxa, sxb)
        zero = jnp.zeros((L,), jnp.int32)

        def zbody(i, _):
            subh[pl.ds(i * L, L)] = zero
            return 0

        lax.fori_loop(0, NBINS, zbody, 0)

        ones = jnp.ones((L,), jnp.int32)
        lane_f = lax.iota(jnp.int32, L).astype(jnp.float32)
        sixteen = jnp.float32(L)

        def src(c):
            s = wid * slabs_per_w + c // ch1_per_slab
            r = (c % ch1_per_slab) * rows1
            return img_hbm.at[s, pl.ds(r, rows1)]

        def xdst(c):
            return idx_hbm.at[pl.ds(wid * (per_w // 2) + c * (chunk1 // 2), chunk1 // 2)]

        pltpu.async_copy(src(0), bufa, sema)
        pltpu.async_copy(src(1), bufb, semb)

        nvp = d2 // L

        def pbody(p, _):
            for b in range(2):
                c = 2 * p + b
                buf, sem = bufs[b], sems[b]
                ix, xsem = ixbufs[b], xsems[b]
                pltpu.make_async_copy(src(c), buf, sem).wait()

                @pl.when(c >= 2)
                def _():
                    pltpu.make_async_copy(ix, xdst(c - 2), xsem).wait()

                @plsc.parallel_loop(0, chunk1 // (2 * L), 1, unroll=8)
                def _(i):
                    q0 = 2 * i
                    q1 = 2 * i + 1
                    va = buf[q0 // nvp, pl.ds((q0 % nvp) * L, L)]
                    vb = buf[q1 // nvp, pl.ds((q1 % nvp) * L, L)]
                    ia = (va * sixteen + lane_f).astype(jnp.int32)
                    ib = (vb * sixteen + lane_f).astype(jnp.int32)
                    plsc.addupdate_scatter(subh, [ia], ones)
                    plsc.addupdate_scatter(subh, [ib], ones)
                    packed = plsc.pack(ia, ib, format=plsc.PackFormat.INTERLEAVED)
                    ix[pl.ds(pl.multiple_of(i * L, L), L)] = plsc.bitcast(
                        packed, jnp.int32
                    )

                pltpu.async_copy(ix, xdst(c), xsem)

                @pl.when(c + 2 < nch1)
                def _():
                    pltpu.async_copy(src(c + 2), buf, sem)

            return 0

        lax.fori_loop(0, nch1 // 2, pbody, 0)
        pltpu.make_async_copy(ixa, xdst(nch1 - 2), sxa).wait()
        pltpu.make_async_copy(ixb, xdst(nch1 - 1), sxb).wait()

        lane_i = lax.iota(jnp.int32, L)

        def rbody(j, _):
            res = jnp.zeros((L,), jnp.int32)
            for k in range(L):
                row = subh[pl.ds((j * L + k) * L, L)]
                s = jnp.sum(row)
                res = jnp.where(lane_i == k, s, res)
            partial[pl.ds(j * L, L)] = res
            return 0

        lax.fori_loop(0, NBINS // L, rbody, 0)
        pltpu.sync_copy(partial, out_hbm.at[pl.ds(wid * NBINS, NBINS)])

    @functools.partial(
        pl.kernel,
        out_type=jax.ShapeDtypeStruct((d0, d1, d2), jnp.float32),
        mesh=mesh,
        compiler_params=_CP,
        scratch_types=[
            pltpu.VMEM((NW * NBINS,), jnp.int32),
            pltpu.VMEM((NBINS * L,), jnp.float32),
            pltpu.VMEM((NBINS,), jnp.float32),
            pltpu.VMEM((chunk2 // 2,), jnp.int32),
            pltpu.VMEM((chunk2 // 2,), jnp.int32),
            pltpu.VMEM((rows2, d2), jnp.float32),
            pltpu.VMEM((rows2, d2), jnp.float32),
            pltpu.SemaphoreType.DMA,
            pltpu.SemaphoreType.DMA,
            pltpu.SemaphoreType.DMA,
            pltpu.SemaphoreType.DMA,
        ],
    )
    def mask_k(idx_hbm, hist_hbm, out_hbm, parts, table2, staging, ina, inb,
               outa, outb, sia, sib, soa, sob):
        wid = lax.axis_index("s") * NC + lax.axis_index("c")
        ibufs, obufs = (ina, inb), (outa, outb)
        isems, osems = (sia, sib), (soa, sob)

        def src(c):
            return idx_hbm.at[pl.ds(wid * (per_w // 2) + c * (chunk2 // 2), chunk2 // 2)]

        def dst(c):
            s = wid * slabs_per_w + c // ch2_per_slab
            r = (c % ch2_per_slab) * rows2
            return out_hbm.at[s, pl.ds(r, rows2)]

        pltpu.async_copy(src(0), ina, sia)
        pltpu.async_copy(src(1), inb, sib)

        pltpu.sync_copy(hist_hbm, parts)
        lane_i = lax.iota(jnp.int32, L)

        def tbody(j, _):
            acc = parts[pl.ds(j * L, L)]
            for w in range(1, NW):
                acc = acc + parts[pl.ds(w * NBINS + j * L, L)]
            lvals = (lane_i + j * L).astype(jnp.float32)
            keep = (acc >= MIN_SZ) & (acc <= MAX_SZ)
            staging[pl.ds(j * L, L)] = jnp.where(keep, lvals, jnp.float32(0.0))
            return 0

        lax.fori_loop(0, NBINS // L, tbody, 0)

        def t2body(j, _):
            t = staging[pl.ds(j * L, L)]
            for k in range(L):
                table2[pl.ds((j * L + k) * L, L)] = jnp.full((L,), t[k], jnp.float32)
            return 0

        lax.fori_loop(0, NBINS // L, t2body, 0)

        nvp = d2 // L

        def pbody(p, _):
            for b in range(2):
                c = 2 * p + b
                ib, ob = ibufs[b], obufs[b]
                isem, osem = isems[b], osems[b]
                pltpu.make_async_copy(src(c), ib, isem).wait()

                @pl.when(c >= 2)
                def _():
                    pltpu.make_async_copy(ob, dst(c - 2), osem).wait()

                @plsc.parallel_loop(0, chunk2 // (2 * L), 1, unroll=16)
                def _(i):
                    packed = plsc.bitcast(
                        ib[pl.ds(pl.multiple_of(i * L, L), L)], jnp.int16
                    )
                    ia, ixb = plsc.unpack(packed, format=plsc.PackFormat.INTERLEAVED)
                    q0 = 2 * i
                    q1 = 2 * i + 1
                    ob[q0 // nvp, pl.ds((q0 % nvp) * L, L)] = plsc.load_gather(
                        table2, [ia]
                    )
                    ob[q1 // nvp, pl.ds((q1 % nvp) * L, L)] = plsc.load_gather(
                        table2, [ixb]
                    )

                pltpu.async_copy(ob, dst(c), osem)

                @pl.when(c + 2 < nch2)
                def _():
                    pltpu.async_copy(src(c + 2), ib, isem)

            return 0

        lax.fori_loop(0, nch2 // 2, pbody, 0)
        pltpu.make_async_copy(outa, dst(nch2 - 2), soa).wait()
        pltpu.make_async_copy(outb, dst(nch2 - 1), sob).wait()

    return hist_k, mask_k


def kernel(image):
    d0, d1, d2 = image.shape
    rows1 = 64
    while d1 % (rows1 * 2) != 0:
        rows1 //= 2
    rows2 = rows1 // 2
    hist_k, mask_k = _build(d0, d1, d2, rows1, rows2)
    parts, idx16 = hist_k(image)
    return mask_k(idx16, parts)


# speedup vs baseline: 1450.4372x; 1.0009x over previous
import functools

import jax
import jax.numpy as jnp
from jax import lax
from jax.experimental import pallas as pl
from jax.experimental.pallas import tpu as pltpu
from jax.experimental.pallas import tpu_sc as plsc

MIN_SZ = 33000
MAX_SZ = 34000
NBINS = 1024
NC, NS, L = 2, 16, 16
NW = NC * NS

_CP = pltpu.CompilerParams(needs_layout_passes=False, use_tc_tiling_on_sc=True)


@functools.lru_cache(maxsize=None)
def _build(d0, d1, d2, rows1, rows2):
    slabs_per_w = d0 // NW
    nch1 = slabs_per_w * (d1 // rows1)
    nch2 = slabs_per_w * (d1 // rows2)
    ch1_per_slab = d1 // rows1
    ch2_per_slab = d1 // rows2
    chunk1 = rows1 * d2
    chunk2 = rows2 * d2
    per_w = slabs_per_w * d1 * d2
    n = d0 * d1 * d2
    mesh = plsc.VectorSubcoreMesh(
        core_axis_name="c", subcore_axis_name="s", num_cores=NC, num_subcores=NS
    )

    @functools.partial(
        pl.kernel,
        out_type=(
            jax.ShapeDtypeStruct((NW * NBINS,), jnp.int32),
            jax.ShapeDtypeStruct((n // 2,), jnp.int32),
        ),
        mesh=mesh,
        compiler_params=_CP,
        scratch_types=[
            pltpu.VMEM((rows1, d2), jnp.float32),
            pltpu.VMEM((rows1, d2), jnp.float32),
            pltpu.VMEM((chunk1 // 2,), jnp.int32),
            pltpu.VMEM((chunk1 // 2,), jnp.int32),
            pltpu.VMEM((NBINS * L,), jnp.int32),
            pltpu.VMEM((NBINS,), jnp.int32),
            pltpu.SemaphoreType.DMA,
            pltpu.SemaphoreType.DMA,
            pltpu.SemaphoreType.DMA,
            pltpu.SemaphoreType.DMA,
        ],
    )
    def hist_k(img_hbm, out_hbm, idx_hbm, bufa, bufb, ixa, ixb, subh, partial,
               sema, semb, sxa, sxb):
        wid = lax.axis_index("s") * NC + lax.axis_index("c")
        bufs = (bufa, bufb)
        ixbufs = (ixa, ixb)
        sems = (sema, semb)
        xsems = (sxa, sxb)
        zero = jnp.zeros((L,), jnp.int32)

        def zbody(i, _):
            subh[pl.ds(i * L, L)] = zero
            return 0

        lax.fori_loop(0, NBINS, zbody, 0)

        ones = jnp.ones((L,), jnp.int32)
        lane_f = lax.iota(jnp.int32, L).astype(jnp.float32)
        sixteen = jnp.float32(L)

        def src(c):
            s = wid * slabs_per_w + c // ch1_per_slab
            r = (c % ch1_per_slab) * rows1
            return img_hbm.at[s, pl.ds(r, rows1)]

        def xdst(c):
            return idx_hbm.at[pl.ds(wid * (per_w // 2) + c * (chunk1 // 2), chunk1 // 2)]

        pltpu.async_copy(src(0), bufa, sema)
        pltpu.async_copy(src(1), bufb, semb)

        nvp = d2 // L

        def pbody(p, _):
            for b in range(2):
                c = 2 * p + b
                buf, sem = bufs[b], sems[b]
                ix, xsem = ixbufs[b], xsems[b]
                pltpu.make_async_copy(src(c), buf, sem).wait()

                @pl.when(c >= 2)
                def _():
                    pltpu.make_async_copy(ix, xdst(c - 2), xsem).wait()

                @plsc.parallel_loop(0, chunk1 // (2 * L), 1, unroll=8)
                def _(i):
                    q0 = 2 * i
                    q1 = 2 * i + 1
                    va = buf[q0 // nvp, pl.ds((q0 % nvp) * L, L)]
                    vb = buf[q1 // nvp, pl.ds((q1 % nvp) * L, L)]
                    ia = (va * sixteen + lane_f).astype(jnp.int32)
                    ib = (vb * sixteen + lane_f).astype(jnp.int32)
                    plsc.addupdate_scatter(subh, [ia], ones)
                    plsc.addupdate_scatter(subh, [ib], ones)
                    packed = plsc.pack(ia, ib, format=plsc.PackFormat.INTERLEAVED)
                    ix[pl.ds(pl.multiple_of(i * L, L), L)] = plsc.bitcast(
                        packed, jnp.int32
                    )

                pltpu.async_copy(ix, xdst(c), xsem)

                @pl.when(c + 2 < nch1)
                def _():
                    pltpu.async_copy(src(c + 2), buf, sem)

            return 0

        lax.fori_loop(0, nch1 // 2, pbody, 0)
        pltpu.make_async_copy(ixa, xdst(nch1 - 2), sxa).wait()
        pltpu.make_async_copy(ixb, xdst(nch1 - 1), sxb).wait()

        lane_i = lax.iota(jnp.int32, L)

        def rbody(j, _):
            res = jnp.zeros((L,), jnp.int32)
            for k in range(L):
                row = subh[pl.ds((j * L + k) * L, L)]
                s = jnp.sum(row)
                res = jnp.where(lane_i == k, s, res)
            partial[pl.ds(j * L, L)] = res
            return 0

        lax.fori_loop(0, NBINS // L, rbody, 0)
        pltpu.sync_copy(partial, out_hbm.at[pl.ds(wid * NBINS, NBINS)])

    @functools.partial(
        pl.kernel,
        out_type=jax.ShapeDtypeStruct((d0, d1, d2), jnp.float32),
        mesh=mesh,
        compiler_params=_CP,
        scratch_types=[
            pltpu.VMEM((NW * NBINS,), jnp.int32),
            pltpu.VMEM((NBINS * L,), jnp.float32),
            pltpu.VMEM((NBINS,), jnp.float32),
            pltpu.VMEM((chunk2 // 2,), jnp.int32),
            pltpu.VMEM((chunk2 // 2,), jnp.int32),
            pltpu.VMEM((rows2, d2), jnp.float32),
            pltpu.VMEM((rows2, d2), jnp.float32),
            pltpu.SemaphoreType.DMA,
            pltpu.SemaphoreType.DMA,
            pltpu.SemaphoreType.DMA,
            pltpu.SemaphoreType.DMA,
        ],
    )
    def mask_k(idx_hbm, hist_hbm, out_hbm, parts, table2, staging, ina, inb,
               outa, outb, sia, sib, soa, sob):
        wid = lax.axis_index("s") * NC + lax.axis_index("c")
        ibufs, obufs = (ina, inb), (outa, outb)
        isems, osems = (sia, sib), (soa, sob)

        def src(c):
            return idx_hbm.at[pl.ds(wid * (per_w // 2) + c * (chunk2 // 2), chunk2 // 2)]

        def dst(c):
            s = wid * slabs_per_w + c // ch2_per_slab
            r = (c % ch2_per_slab) * rows2
            return out_hbm.at[s, pl.ds(r, rows2)]

        pltpu.async_copy(src(0), ina, sia)
        pltpu.async_copy(src(1), inb, sib)

        pltpu.sync_copy(hist_hbm, parts)
        lane_i = lax.iota(jnp.int32, L)

        def tbody(j, _):
            acc = parts[pl.ds(j * L, L)]
            for w in range(1, NW):
                acc = acc + parts[pl.ds(w * NBINS + j * L, L)]
            lvals = (lane_i + j * L).astype(jnp.float32)
            keep = (acc >= MIN_SZ) & (acc <= MAX_SZ)
            staging[pl.ds(j * L, L)] = jnp.where(keep, lvals, jnp.float32(0.0))
            return 0

        lax.fori_loop(0, NBINS // L, tbody, 0)

        def t2body(j, _):
            t = staging[pl.ds(j * L, L)]
            for k in range(L):
                table2[pl.ds((j * L + k) * L, L)] = jnp.full((L,), t[k], jnp.float32)
            return 0

        lax.fori_loop(0, NBINS // L, t2body, 0)

        nvp = d2 // L

        def pbody(p, _):
            for b in range(2):
                c = 2 * p + b
                ib, ob = ibufs[b], obufs[b]
                isem, osem = isems[b], osems[b]
                pltpu.make_async_copy(src(c), ib, isem).wait()

                @pl.when(c >= 2)
                def _():
                    pltpu.make_async_copy(ob, dst(c - 2), osem).wait()

                @plsc.parallel_loop(0, chunk2 // (2 * L), 1, unroll=8)
                def _(i):
                    packed = plsc.bitcast(
                        ib[pl.ds(pl.multiple_of(i * L, L), L)], jnp.int16
                    )
                    ia, ixb = plsc.unpack(packed, format=plsc.PackFormat.INTERLEAVED)
                    q0 = 2 * i
                    q1 = 2 * i + 1
                    ob[q0 // nvp, pl.ds((q0 % nvp) * L, L)] = plsc.load_gather(
                        table2, [ia]
                    )
                    ob[q1 // nvp, pl.ds((q1 % nvp) * L, L)] = plsc.load_gather(
                        table2, [ixb]
                    )

                pltpu.async_copy(ob, dst(c), osem)

                @pl.when(c + 2 < nch2)
                def _():
                    pltpu.async_copy(src(c + 2), ib, isem)

            return 0

        lax.fori_loop(0, nch2 // 2, pbody, 0)
        pltpu.make_async_copy(outa, dst(nch2 - 2), soa).wait()
        pltpu.make_async_copy(outb, dst(nch2 - 1), sob).wait()

    return hist_k, mask_k


def kernel(image):
    d0, d1, d2 = image.shape
    rows1 = 64
    while d1 % (rows1 * 2) != 0:
        rows1 //= 2
    rows2 = rows1 // 2
    hist_k, mask_k = _build(d0, d1, d2, rows1, rows2)
    parts, idx16 = hist_k(image)
    return mask_k(idx16, parts)
